# Initial kernel scaffold; baseline (speedup 1.0000x reference)
import numpy as np
import jax
import jax.numpy as jnp
from jax.experimental import pallas as pl
from jax.experimental.pallas import tpu as pltpu

_F32 = jnp.float32


def _np_row_tap(h, r, k):
    a = np.zeros((h, r, k), np.float32)
    for rr in range(r):
        for i in range(k):
            if rr + i < h:
                a[rr + i, rr, i] = 1.0
    return a


def _np_col_tap(w, p, k):
    c = np.zeros((w, p, 2, k), np.float32)
    for pp in range(p):
        for a in range(2):
            for j in range(k):
                if 2 * pp + a + j < w:
                    c[2 * pp + a + j, pp, a, j] = 1.0
    return c


def _pad_last(x, target):
    return jnp.pad(x, [(0, 0)] * (x.ndim - 1) + [(0, target - x.shape[-1])])


def _build_tables(conv1_w, conv1_b, conv2_w, conv2_b,
                  fc1_w, fc1_b, fc2_w, fc2_b, fc3_w, fc3_b):
    f = _F32
    w1 = conv1_w.astype(f)[:, 0]
    w2 = conv2_w.astype(f)

    A1 = jnp.asarray(_np_row_tap(8, 4, 5))
    B1 = jnp.asarray(_np_col_tap(32, 14, 2, 5))
    W1 = jnp.einsum("hri,wpaj,oij->hwraop", A1, B1, w1)
    W1 = W1.reshape(256, 4, 2, 84)
    W1 = _pad_last(W1, 128).reshape(256, 1024)
    b1 = jnp.repeat(conv1_b.astype(f), 14)
    b1 = _pad_last(jnp.broadcast_to(b1, (4, 2, 84)), 128).reshape(1, 1024)

    C2 = jnp.asarray(_np_col_tap(14, 5, 2, 5))
    W2 = jnp.einsum("upaj,ocij->icuaop", C2, w2)
    W2 = W2.reshape(5, 84, 2, 80)
    W2 = jnp.pad(W2, ((0, 0), (0, 44), (0, 0), (0, 48))).reshape(640, 256)
    b2 = jnp.repeat(conv2_b.astype(f), 5)
    b2 = _pad_last(jnp.broadcast_to(b2, (2, 80)), 128).reshape(1, 256)

    WF1 = (fc1_w.astype(f).reshape(120, 16, 5, 5)
           .transpose(2, 1, 3, 0).reshape(5, 80, 120))
    WF1 = jnp.pad(WF1, ((0, 0), (0, 48), (0, 8))).reshape(640, 128)
    bF1 = _pad_last(fc1_b.astype(f)[None, :], 128)

    WF2 = jnp.pad(fc2_w.astype(f).T, ((0, 8), (0, 44)))
    bF2 = _pad_last(fc2_b.astype(f)[None, :], 128)
    WF3 = jnp.pad(fc3_w.astype(f).T, ((0, 44), (0, 118)))
    bF3 = _pad_last(fc3_b.astype(f)[None, :], 128)
    return W1, b1, W2, b2, WF1, bF1, WF2, bF2, WF3, bF3


def _net_kernel(x_ref, w1_ref, b1_ref, w2_ref, b2_ref, wf1_ref, bf1_ref,
                wf2_ref, bf2_ref, wf3_ref, bf3_ref, out_ref):
    f = _F32
    x = x_ref[...]
    w1 = w1_ref[...]
    b1 = b1_ref[...]

    p1_rows = []
    for c in range(7):
        y = jnp.dot(x[:, 128 * c:128 * c + 256], w1,
                    preferred_element_type=f) + b1
        m = [jnp.maximum(y[:, g * 256:g * 256 + 128],
                         y[:, g * 256 + 128:g * 256 + 256])
             for g in range(4)]
        p1_rows.append(jnp.maximum(jnp.maximum(m[0], m[1]), 0.0))
        p1_rows.append(jnp.maximum(jnp.maximum(m[2], m[3]), 0.0))
    p1 = jnp.concatenate(p1_rows, axis=1)

    w2 = w2_ref[...]
    b2 = b2_ref[...]
    cm = []
    for r in range(10):
        y2 = jnp.dot(p1[:, 128 * r:128 * r + 640], w2,
                     preferred_element_type=f) + b2
        cm.append(jnp.maximum(y2[:, :128], y2[:, 128:]))
    p2_rows = [jnp.maximum(jnp.maximum(cm[2 * q], cm[2 * q + 1]), 0.0)
               for q in range(5)]
    p2 = jnp.concatenate(p2_rows, axis=1)

    h1 = jnp.maximum(jnp.dot(p2, wf1_ref[...], preferred_element_type=f)
                     + bf1_ref[...], 0.0)
    h2 = jnp.maximum(jnp.dot(h1, wf2_ref[...], preferred_element_type=f)
                     + bf2_ref[...], 0.0)
    out_ref[...] = (jnp.dot(h2, wf3_ref[...], preferred_element_type=f)
                    + bf3_ref[...])


def _const_spec(a):
    zeros = (0,) * a.ndim
    return pl.BlockSpec(a.shape, lambda b, _z=zeros: _z)


def kernel(x, conv1_w, conv1_b, conv2_w, conv2_b, fc1_w, fc1_b,
           fc2_w, fc2_b, fc3_w, fc3_b):
    B = x.shape[0]
    tables = _build_tables(conv1_w, conv1_b, conv2_w, conv2_b,
                           fc1_w, fc1_b, fc2_w, fc2_b, fc3_w, fc3_b)
    x2 = x.astype(_F32).reshape(B, 1024)

    for bb in (512, 256, 128, 64, 32, 16, 8, 1):
        if B % bb == 0:
            BB = bb
            break

    in_specs = [pl.BlockSpec((BB, 1024), lambda b: (b, 0))]
    in_specs += [_const_spec(a) for a in tables]

    out = pl.pallas_call(
        _net_kernel,
        out_shape=jax.ShapeDtypeStruct((B, 128), _F32),
        grid=(B // BB,),
        in_specs=in_specs,
        out_specs=pl.BlockSpec((BB, 128), lambda b: (b, 0)),
        compiler_params=pltpu.CompilerParams(
            dimension_semantics=("parallel",)),
    )(x2, *tables)
    return out[:, :10]


# speedup vs baseline: 72.5040x; 72.5040x over previous
import numpy as np
import jax
import jax.numpy as jnp
from jax.experimental import pallas as pl
from jax.experimental.pallas import tpu as pltpu

_F32 = jnp.float32


def _np_row_tap(h, r, k):
    a = np.zeros((h, r, k), np.float32)
    for rr in range(r):
        for i in range(k):
            if rr + i < h:
                a[rr + i, rr, i] = 1.0
    return a


def _np_col_tap(w, p, k):
    c = np.zeros((w, p, 2, k), np.float32)
    for pp in range(p):
        for a in range(2):
            for j in range(k):
                if 2 * pp + a + j < w:
                    c[2 * pp + a + j, pp, a, j] = 1.0
    return c


def _pad_last(x, target):
    return jnp.pad(x, [(0, 0)] * (x.ndim - 1) + [(0, target - x.shape[-1])])


def _build_tables(conv1_w, conv1_b, conv2_w, conv2_b,
                  fc1_w, fc1_b, fc2_w, fc2_b, fc3_w, fc3_b):
    f = _F32
    w1 = conv1_w.astype(f)[:, 0]
    w2 = conv2_w.astype(f)

    A1 = jnp.asarray(_np_row_tap(8, 4, 5))
    B1 = jnp.asarray(_np_col_tap(32, 14, 5))
    W1 = jnp.einsum("hri,wpaj,oij->hwraop", A1, B1, w1)
    W1 = W1.reshape(256, 4, 2, 84)
    W1 = _pad_last(W1, 128).reshape(256, 1024)
    b1 = jnp.repeat(conv1_b.astype(f), 14)
    b1 = _pad_last(jnp.broadcast_to(b1, (4, 2, 84)), 128).reshape(1, 1024)

    C2 = jnp.asarray(_np_col_tap(14, 5, 5))
    W2 = jnp.einsum("upaj,ocij->icuaop", C2, w2)
    W2 = W2.reshape(5, 84, 2, 80)
    W2 = jnp.pad(W2, ((0, 0), (0, 44), (0, 0), (0, 48))).reshape(640, 256)
    b2 = jnp.repeat(conv2_b.astype(f), 5)
    b2 = _pad_last(jnp.broadcast_to(b2, (2, 80)), 128).reshape(1, 256)

    WF1 = (fc1_w.astype(f).reshape(120, 16, 5, 5)
           .transpose(2, 1, 3, 0).reshape(5, 80, 120))
    WF1 = jnp.pad(WF1, ((0, 0), (0, 48), (0, 8))).reshape(640, 128)
    bF1 = _pad_last(fc1_b.astype(f)[None, :], 128)

    WF2 = jnp.pad(fc2_w.astype(f).T, ((0, 8), (0, 44)))
    bF2 = _pad_last(fc2_b.astype(f)[None, :], 128)
    WF3 = jnp.pad(fc3_w.astype(f).T, ((0, 44), (0, 118)))
    bF3 = _pad_last(fc3_b.astype(f)[None, :], 128)
    return W1, b1, W2, b2, WF1, bF1, WF2, bF2, WF3, bF3


def _net_kernel(x_ref, w1_ref, b1_ref, w2_ref, b2_ref, wf1_ref, bf1_ref,
                wf2_ref, bf2_ref, wf3_ref, bf3_ref, out_ref):
    f = _F32
    x = x_ref[...]
    w1 = w1_ref[...]
    b1 = b1_ref[...]

    p1_rows = []
    for c in range(7):
        y = jnp.dot(x[:, 128 * c:128 * c + 256], w1,
                    preferred_element_type=f) + b1
        m = [jnp.maximum(y[:, g * 256:g * 256 + 128],
                         y[:, g * 256 + 128:g * 256 + 256])
             for g in range(4)]
        p1_rows.append(jnp.maximum(jnp.maximum(m[0], m[1]), 0.0))
        p1_rows.append(jnp.maximum(jnp.maximum(m[2], m[3]), 0.0))
    p1 = jnp.concatenate(p1_rows, axis=1)

    w2 = w2_ref[...]
    b2 = b2_ref[...]
    cm = []
    for r in range(10):
        y2 = jnp.dot(p1[:, 128 * r:128 * r + 640], w2,
                     preferred_element_type=f) + b2
        cm.append(jnp.maximum(y2[:, :128], y2[:, 128:]))
    p2_rows = [jnp.maximum(jnp.maximum(cm[2 * q], cm[2 * q + 1]), 0.0)
               for q in range(5)]
    p2 = jnp.concatenate(p2_rows, axis=1)

    h1 = jnp.maximum(jnp.dot(p2, wf1_ref[...], preferred_element_type=f)
                     + bf1_ref[...], 0.0)
    h2 = jnp.maximum(jnp.dot(h1, wf2_ref[...], preferred_element_type=f)
                     + bf2_ref[...], 0.0)
    out_ref[...] = (jnp.dot(h2, wf3_ref[...], preferred_element_type=f)
                    + bf3_ref[...])


def _const_spec(a):
    zeros = (0,) * a.ndim
    return pl.BlockSpec(a.shape, lambda b, _z=zeros: _z)


def kernel(x, conv1_w, conv1_b, conv2_w, conv2_b, fc1_w, fc1_b,
           fc2_w, fc2_b, fc3_w, fc3_b):
    B = x.shape[0]
    tables = _build_tables(conv1_w, conv1_b, conv2_w, conv2_b,
                           fc1_w, fc1_b, fc2_w, fc2_b, fc3_w, fc3_b)
    x2 = x.astype(_F32).reshape(B, 1024)

    for bb in (512, 256, 128, 64, 32, 16, 8, 1):
        if B % bb == 0:
            BB = bb
            break

    in_specs = [pl.BlockSpec((BB, 1024), lambda b: (b, 0))]
    in_specs += [_const_spec(a) for a in tables]

    out = pl.pallas_call(
        _net_kernel,
        out_shape=jax.ShapeDtypeStruct((B, 128), _F32),
        grid=(B // BB,),
        in_specs=in_specs,
        out_specs=pl.BlockSpec((BB, 128), lambda b: (b, 0)),
        compiler_params=pltpu.CompilerParams(
            dimension_semantics=("parallel",)),
    )(x2, *tables)
    return out[:, :10]


# speedup vs baseline: 73.6240x; 1.0154x over previous
import numpy as np
import jax
import jax.numpy as jnp
from jax.experimental import pallas as pl
from jax.experimental.pallas import tpu as pltpu

_F32 = jnp.float32


def _np_row_tap(h, r, k):
    a = np.zeros((h, r, k), np.float32)
    for rr in range(r):
        for i in range(k):
            if rr + i < h:
                a[rr + i, rr, i] = 1.0
    return a


def _np_col_tap(w, p, k):
    c = np.zeros((w, p, 2, k), np.float32)
    for pp in range(p):
        for a in range(2):
            for j in range(k):
                if 2 * pp + a + j < w:
                    c[2 * pp + a + j, pp, a, j] = 1.0
    return c


def _pad_last(x, target):
    return jnp.pad(x, [(0, 0)] * (x.ndim - 1) + [(0, target - x.shape[-1])])


def _build_tables(conv1_w, conv1_b, conv2_w, conv2_b,
                  fc1_w, fc1_b, fc2_w, fc2_b, fc3_w, fc3_b):
    f = _F32
    w1 = conv1_w.astype(f)[:, 0]
    w2 = conv2_w.astype(f)

    A1 = jnp.asarray(_np_row_tap(8, 4, 5))
    B1 = jnp.asarray(_np_col_tap(32, 14, 5))
    W1 = jnp.einsum("hri,wpaj,oij->hwraop", A1, B1, w1)
    W1 = W1.reshape(256, 4, 2, 84)
    W1 = _pad_last(W1, 128).reshape(256, 1024)
    b1 = jnp.repeat(conv1_b.astype(f), 14)
    b1 = _pad_last(jnp.broadcast_to(b1, (4, 2, 84)), 128).reshape(1, 1024)

    C2 = jnp.asarray(_np_col_tap(14, 5, 5))
    W2 = jnp.einsum("upaj,ocij->icuaop", C2, w2)
    W2 = W2.reshape(5, 84, 2, 80)
    W2 = jnp.pad(W2, ((0, 0), (0, 44), (0, 0), (0, 48))).reshape(640, 256)
    b2 = jnp.repeat(conv2_b.astype(f), 5)
    b2 = _pad_last(jnp.broadcast_to(b2, (2, 80)), 128).reshape(1, 256)

    WF1 = (fc1_w.astype(f).reshape(120, 16, 5, 5)
           .transpose(2, 1, 3, 0).reshape(5, 80, 120))
    WF1 = jnp.pad(WF1, ((0, 0), (0, 48), (0, 8))).reshape(640, 128)
    bF1 = _pad_last(fc1_b.astype(f)[None, :], 128)

    WF2 = jnp.pad(fc2_w.astype(f).T, ((0, 8), (0, 44)))
    bF2 = _pad_last(fc2_b.astype(f)[None, :], 128)
    WF3 = jnp.pad(fc3_w.astype(f).T, ((0, 44), (0, 118)))
    bF3 = _pad_last(fc3_b.astype(f)[None, :], 128)
    return W1, b1, W2, b2, WF1, bF1, WF2, bF2, WF3, bF3


def _net_kernel(x_ref, w1_ref, b1_ref, w2_ref, b2_ref, wf1_ref, bf1_ref,
                wf2_ref, bf2_ref, wf3_ref, bf3_ref, out_ref):
    f = _F32
    x = x_ref[...]
    w1 = w1_ref[...]
    b1 = b1_ref[...]

    p1_rows = []
    for c in range(7):
        y = jnp.dot(x[:, 128 * c:128 * c + 256], w1,
                    preferred_element_type=f) + b1
        m = [jnp.maximum(y[:, g * 256:g * 256 + 128],
                         y[:, g * 256 + 128:g * 256 + 256])
             for g in range(4)]
        p1_rows.append(jnp.maximum(jnp.maximum(m[0], m[1]), 0.0))
        p1_rows.append(jnp.maximum(jnp.maximum(m[2], m[3]), 0.0))
    p1 = jnp.concatenate(p1_rows, axis=1)

    w2 = w2_ref[...]
    b2 = b2_ref[...]
    cm = []
    for r in range(10):
        y2 = jnp.dot(p1[:, 128 * r:128 * r + 640], w2,
                     preferred_element_type=f) + b2
        cm.append(jnp.maximum(y2[:, :128], y2[:, 128:]))
    p2_rows = [jnp.maximum(jnp.maximum(cm[2 * q], cm[2 * q + 1]), 0.0)
               for q in range(5)]
    p2 = jnp.concatenate(p2_rows, axis=1)

    h1 = jnp.maximum(jnp.dot(p2, wf1_ref[...], preferred_element_type=f)
                     + bf1_ref[...], 0.0)
    h2 = jnp.maximum(jnp.dot(h1, wf2_ref[...], preferred_element_type=f)
                     + bf2_ref[...], 0.0)
    out_ref[...] = (jnp.dot(h2, wf3_ref[...], preferred_element_type=f)
                    + bf3_ref[...])


def _const_spec(a):
    zeros = (0,) * a.ndim
    return pl.BlockSpec(a.shape, lambda b, _z=zeros: _z)


def kernel(x, conv1_w, conv1_b, conv2_w, conv2_b, fc1_w, fc1_b,
           fc2_w, fc2_b, fc3_w, fc3_b):
    B = x.shape[0]
    tables = _build_tables(conv1_w, conv1_b, conv2_w, conv2_b,
                           fc1_w, fc1_b, fc2_w, fc2_b, fc3_w, fc3_b)
    x2 = x.astype(_F32).reshape(B, 1024)

    for bb in (1024, 256, 128, 64, 32, 16, 8, 1):
        if B % bb == 0:
            BB = bb
            break

    in_specs = [pl.BlockSpec((BB, 1024), lambda b: (b, 0))]
    in_specs += [_const_spec(a) for a in tables]

    out = pl.pallas_call(
        _net_kernel,
        out_shape=jax.ShapeDtypeStruct((B, 128), _F32),
        grid=(B // BB,),
        in_specs=in_specs,
        out_specs=pl.BlockSpec((BB, 128), lambda b: (b, 0)),
        compiler_params=pltpu.CompilerParams(
            dimension_semantics=("parallel",)),
    )(x2, *tables)
    return out[:, :10]


# speedup vs baseline: 76.3346x; 1.0368x over previous
import numpy as np
import jax
import jax.numpy as jnp
from jax.experimental import pallas as pl
from jax.experimental.pallas import tpu as pltpu

_F32 = jnp.float32


def _np_row_tap(h, r, k):
    a = np.zeros((h, r, k), np.float32)
    for rr in range(r):
        for i in range(k):
            if rr + i < h:
                a[rr + i, rr, i] = 1.0
    return a


def _np_col_tap(w, p, k):
    c = np.zeros((w, p, 2, k), np.float32)
    for pp in range(p):
        for a in range(2):
            for j in range(k):
                if 2 * pp + a + j < w:
                    c[2 * pp + a + j, pp, a, j] = 1.0
    return c


def _pad_last(x, target):
    return jnp.pad(x, [(0, 0)] * (x.ndim - 1) + [(0, target - x.shape[-1])])


def _np_onehot(src, dst, fn):
    m = np.zeros((src, dst), np.float32)
    for s in range(src):
        m[s, fn(s)] = 1.0
    return m


def _build_tables(conv1_w, conv1_b, conv2_w, conv2_b,
                  fc1_w, fc1_b, fc2_w, fc2_b, fc3_w, fc3_b):
    f = _F32
    w1 = conv1_w.astype(f)[:, 0]
    w2 = conv2_w.astype(f)

    A1 = jnp.asarray(_np_row_tap(8, 4, 5))
    B1 = jnp.asarray(_np_col_tap(32, 14, 5))
    D1o = jnp.asarray(_np_onehot(84, 128, lambda s: s).reshape(6, 14, 128))
    W1 = jnp.einsum("hri,wqaj,oij,oql->hwral", A1, B1, w1, D1o)
    W1 = W1.reshape(256, 1024)
    P1 = jnp.asarray(
        _np_onehot(84, 128, lambda s: s).reshape(6, 14, 128).sum(1))
    b1 = jnp.tile(jnp.dot(conv1_b.astype(f)[None, :], P1), (1, 8))

    C2 = jnp.asarray(_np_col_tap(14, 5, 5))
    E2 = jnp.asarray(_np_onehot(84, 128, lambda s: s).reshape(6, 14, 128))
    G2 = jnp.asarray(_np_onehot(80, 128, lambda s: s).reshape(16, 5, 128))
    W2 = jnp.einsum("uqaj,ocij,cuk,oql->ikal", C2, w2, E2, G2)
    W2 = W2.reshape(640, 256)
    P2 = jnp.asarray(_np_onehot(80, 128, lambda s: s).reshape(16, 5, 128).sum(1))
    b2 = jnp.tile(jnp.dot(conv2_b.astype(f)[None, :], P2), (1, 2))

    S1 = jnp.asarray(
        _np_onehot(400, 5 * 128,
                   lambda s: (s % 25) // 5 * 128 + (s // 25) * 5 + s % 5))
    N1 = jnp.asarray(_np_onehot(120, 128, lambda s: s))
    WF1 = jnp.einsum("nk,kr,nl->rl", fc1_w.astype(f), S1, N1)
    WF1 = WF1.reshape(640, 128)
    bF1 = jnp.dot(fc1_b.astype(f)[None, :], N1)

    N2 = jnp.asarray(_np_onehot(84, 128, lambda s: s))
    WF2 = jnp.einsum("nk,kr,nl->rl", fc2_w.astype(f), N1, N2)
    bF2 = jnp.dot(fc2_b.astype(f)[None, :], N2)
    N3 = jnp.asarray(_np_onehot(10, 128, lambda s: s))
    WF3 = jnp.einsum("nk,kr,nl->rl", fc3_w.astype(f), N2, N3)
    bF3 = jnp.dot(fc3_b.astype(f)[None, :], N3)
    return W1, b1, W2, b2, WF1, bF1, WF2, bF2, WF3, bF3


def _net_kernel(x_ref, w1_ref, b1_ref, w2_ref, b2_ref, wf1_ref, bf1_ref,
                wf2_ref, bf2_ref, wf3_ref, bf3_ref, out_ref):
    f = _F32
    x = x_ref[...]
    w1 = w1_ref[...]
    b1 = b1_ref[...]

    p1_rows = []
    for c in range(7):
        y = jnp.dot(x[:, 128 * c:128 * c + 256], w1,
                    preferred_element_type=f) + b1
        m = [jnp.maximum(y[:, g * 256:g * 256 + 128],
                         y[:, g * 256 + 128:g * 256 + 256])
             for g in range(4)]
        p1_rows.append(jnp.maximum(jnp.maximum(m[0], m[1]), 0.0))
        p1_rows.append(jnp.maximum(jnp.maximum(m[2], m[3]), 0.0))
    p1 = jnp.concatenate(p1_rows, axis=1)

    w2 = w2_ref[...]
    b2 = b2_ref[...]
    cm = []
    for r in range(10):
        y2 = jnp.dot(p1[:, 128 * r:128 * r + 640], w2,
                     preferred_element_type=f) + b2
        cm.append(jnp.maximum(y2[:, :128], y2[:, 128:]))
    p2_rows = [jnp.maximum(jnp.maximum(cm[2 * q], cm[2 * q + 1]), 0.0)
               for q in range(5)]
    p2 = jnp.concatenate(p2_rows, axis=1)

    h1 = jnp.maximum(jnp.dot(p2, wf1_ref[...], preferred_element_type=f)
                     + bf1_ref[...], 0.0)
    h2 = jnp.maximum(jnp.dot(h1, wf2_ref[...], preferred_element_type=f)
                     + bf2_ref[...], 0.0)
    out_ref[...] = (jnp.dot(h2, wf3_ref[...], preferred_element_type=f)
                    + bf3_ref[...])


def _const_spec(a):
    zeros = (0,) * a.ndim
    return pl.BlockSpec(a.shape, lambda b, _z=zeros: _z)


def kernel(x, conv1_w, conv1_b, conv2_w, conv2_b, fc1_w, fc1_b,
           fc2_w, fc2_b, fc3_w, fc3_b):
    B = x.shape[0]
    tables = _build_tables(conv1_w, conv1_b, conv2_w, conv2_b,
                           fc1_w, fc1_b, fc2_w, fc2_b, fc3_w, fc3_b)
    x2 = x.astype(_F32).reshape(B, 1024)

    for bb in (1024, 256, 128, 64, 32, 16, 8, 1):
        if B % bb == 0:
            BB = bb
            break

    in_specs = [pl.BlockSpec((BB, 1024), lambda b: (b, 0))]
    in_specs += [_const_spec(a) for a in tables]

    out = pl.pallas_call(
        _net_kernel,
        out_shape=jax.ShapeDtypeStruct((B, 128), _F32),
        grid=(B // BB,),
        in_specs=in_specs,
        out_specs=pl.BlockSpec((BB, 128), lambda b: (b, 0)),
        compiler_params=pltpu.CompilerParams(
            dimension_semantics=("parallel",)),
    )(x2, *tables)
    return out[:, :10]


# speedup vs baseline: 79.1840x; 1.0373x over previous
import numpy as np
import jax
import jax.numpy as jnp
from jax.experimental import pallas as pl
from jax.experimental.pallas import tpu as pltpu

_F32 = jnp.float32


def _np_row_tap(h, r, k):
    a = np.zeros((h, r, k), np.float32)
    for rr in range(r):
        for i in range(k):
            if rr + i < h:
                a[rr + i, rr, i] = 1.0
    return a


def _np_col_tap(w, p, k):
    c = np.zeros((w, p, 2, k), np.float32)
    for pp in range(p):
        for a in range(2):
            for j in range(k):
                if 2 * pp + a + j < w:
                    c[2 * pp + a + j, pp, a, j] = 1.0
    return c


def _pad_last(x, target):
    return jnp.pad(x, [(0, 0)] * (x.ndim - 1) + [(0, target - x.shape[-1])])


def _np_onehot(src, dst, fn):
    m = np.zeros((src, dst), np.float32)
    for s in range(src):
        m[s, fn(s)] = 1.0
    return m


def _build_tables(conv1_w, conv1_b, conv2_w, conv2_b,
                  fc1_w, fc1_b, fc2_w, fc2_b, fc3_w, fc3_b):
    f = _F32
    w1 = conv1_w.astype(f)[:, 0]
    w2 = conv2_w.astype(f)

    A1 = jnp.asarray(_np_row_tap(8, 4, 5))
    B1 = jnp.asarray(_np_col_tap(32, 14, 5))
    D1o = jnp.asarray(_np_onehot(84, 128, lambda s: s).reshape(6, 14, 128))
    W1 = jnp.einsum("hri,wqaj,oij,oql->hwral", A1, B1, w1, D1o)
    W1 = W1.reshape(256, 1024)

    C2 = jnp.asarray(_np_col_tap(14, 5, 5))
    E2 = jnp.asarray(_np_onehot(84, 128, lambda s: s).reshape(6, 14, 128))
    G2 = jnp.asarray(_np_onehot(80, 128, lambda s: s).reshape(16, 5, 128))
    W2 = jnp.einsum("uqaj,ocij,cuk,oql->ikal", C2, w2, E2, G2)
    W2 = W2.reshape(640, 256)

    S1 = jnp.asarray(
        _np_onehot(400, 5 * 128,
                   lambda s: (s % 25) // 5 * 128 + (s // 25) * 5 + s % 5))
    N1 = jnp.asarray(_np_onehot(120, 128, lambda s: s))
    WF1 = jnp.einsum("nk,kr,nl->rl", fc1_w.astype(f), S1, N1)

    N2 = jnp.asarray(_np_onehot(84, 128, lambda s: s))
    WF2 = jnp.einsum("nk,kr,nl->rl", fc2_w.astype(f), N1, N2)
    N3 = jnp.asarray(_np_onehot(10, 128, lambda s: s))
    WF3 = jnp.einsum("nk,kr,nl->rl", fc3_w.astype(f), N2, N3)
    WF23 = jnp.concatenate([WF2, WF3], axis=0)

    pb = np.zeros((236, 2 * 1024), np.float32)
    for o in range(6):
        for g in range(8):
            pb[o, g * 128 + o * 14:g * 128 + o * 14 + 14] = 1.0
    for o in range(16):
        for a in range(2):
            pb[6 + o, 1024 + a * 128 + o * 5:1024 + a * 128 + o * 5 + 5] = 1.0
    pb[np.arange(22, 142), 1024 + 256 + np.arange(120)] = 1.0
    pb[np.arange(142, 226), 1024 + 384 + np.arange(84)] = 1.0
    pb[np.arange(226, 236), 1024 + 512 + np.arange(10)] = 1.0
    bcat = jnp.concatenate([conv1_b.astype(f), conv2_b.astype(f),
                            fc1_b.astype(f), fc2_b.astype(f),
                            fc3_b.astype(f)])
    ball = jnp.dot(bcat[None, :], jnp.asarray(pb)).reshape(2, 1024)
    return W1, W2, WF1, WF23, ball


def _net_kernel(x_ref, w1_ref, w2_ref, wf1_ref, wf23_ref, ball_ref, out_ref):
    f = _F32
    x = x_ref[...]
    w1 = w1_ref[...]
    ball = ball_ref[...]
    b1 = ball[0:1, :]

    p1_rows = []
    for c in range(7):
        y = jnp.dot(x[:, 128 * c:128 * c + 256], w1,
                    preferred_element_type=f) + b1
        m = [jnp.maximum(y[:, g * 256:g * 256 + 128],
                         y[:, g * 256 + 128:g * 256 + 256])
             for g in range(4)]
        p1_rows.append(jnp.maximum(jnp.maximum(m[0], m[1]), 0.0))
        p1_rows.append(jnp.maximum(jnp.maximum(m[2], m[3]), 0.0))
    p1 = jnp.concatenate(p1_rows, axis=1)

    w2 = w2_ref[...]
    b2 = ball[1:2, 0:256]
    cm = []
    for r in range(10):
        y2 = jnp.dot(p1[:, 128 * r:128 * r + 640], w2,
                     preferred_element_type=f) + b2
        cm.append(jnp.maximum(y2[:, :128], y2[:, 128:]))
    p2_rows = [jnp.maximum(jnp.maximum(cm[2 * q], cm[2 * q + 1]), 0.0)
               for q in range(5)]
    p2 = jnp.concatenate(p2_rows, axis=1)

    h1 = jnp.maximum(jnp.dot(p2, wf1_ref[...], preferred_element_type=f)
                     + ball[1:2, 256:384], 0.0)
    h2 = jnp.maximum(jnp.dot(h1, wf23_ref[0:128, :], preferred_element_type=f)
                     + ball[1:2, 384:512], 0.0)
    out_ref[...] = (jnp.dot(h2, wf23_ref[128:256, :], preferred_element_type=f)
                    + ball[1:2, 512:640])


def _const_spec(a):
    zeros = (0,) * a.ndim
    return pl.BlockSpec(a.shape, lambda b, _z=zeros: _z)


def kernel(x, conv1_w, conv1_b, conv2_w, conv2_b, fc1_w, fc1_b,
           fc2_w, fc2_b, fc3_w, fc3_b):
    B = x.shape[0]
    tables = _build_tables(conv1_w, conv1_b, conv2_w, conv2_b,
                           fc1_w, fc1_b, fc2_w, fc2_b, fc3_w, fc3_b)
    x2 = x.astype(_F32).reshape(B, 1024)

    for bb in (1024, 256, 128, 64, 32, 16, 8, 1):
        if B % bb == 0:
            BB = bb
            break

    in_specs = [pl.BlockSpec((BB, 1024), lambda b: (b, 0))]
    in_specs += [_const_spec(a) for a in tables]

    out = pl.pallas_call(
        _net_kernel,
        out_shape=jax.ShapeDtypeStruct((B, 128), _F32),
        grid=(B // BB,),
        in_specs=in_specs,
        out_specs=pl.BlockSpec((BB, 128), lambda b: (b, 0)),
        compiler_params=pltpu.CompilerParams(
            dimension_semantics=("parallel",)),
    )(x2, *tables)
    return out[:, :10]


# speedup vs baseline: 79.2238x; 1.0005x over previous
import numpy as np
import jax
import jax.numpy as jnp
from jax.experimental import pallas as pl
from jax.experimental.pallas import tpu as pltpu

_F32 = jnp.float32


def _np_row_tap(h, r, k):
    a = np.zeros((h, r, k), np.float32)
    for rr in range(r):
        for i in range(k):
            if rr + i < h:
                a[rr + i, rr, i] = 1.0
    return a


def _np_col_tap(w, p, k):
    c = np.zeros((w, p, 2, k), np.float32)
    for pp in range(p):
        for a in range(2):
            for j in range(k):
                if 2 * pp + a + j < w:
                    c[2 * pp + a + j, pp, a, j] = 1.0
    return c


def _pad_last(x, target):
    return jnp.pad(x, [(0, 0)] * (x.ndim - 1) + [(0, target - x.shape[-1])])


def _np_onehot(src, dst, fn):
    m = np.zeros((src, dst), np.float32)
    for s in range(src):
        m[s, fn(s)] = 1.0
    return m


def _build_tables(conv1_w, conv1_b, conv2_w, conv2_b,
                  fc1_w, fc1_b, fc2_w, fc2_b, fc3_w, fc3_b):
    f = _F32
    w1 = conv1_w.astype(f)[:, 0]
    w2 = conv2_w.astype(f)

    A1 = jnp.asarray(_np_row_tap(8, 4, 5))
    B1 = jnp.asarray(_np_col_tap(32, 14, 5))
    D1o = jnp.asarray(_np_onehot(84, 128, lambda s: s).reshape(6, 14, 128))
    W1 = jnp.einsum("hri,wqaj,oij,oql->hwral", A1, B1, w1, D1o)
    W1 = W1.reshape(256, 1024)

    C2 = jnp.asarray(_np_col_tap(14, 5, 5))
    E2 = jnp.asarray(_np_onehot(84, 128, lambda s: s).reshape(6, 14, 128))
    G2 = jnp.asarray(_np_onehot(80, 128, lambda s: s).reshape(16, 5, 128))
    W2 = jnp.einsum("uqaj,ocij,cuk,oql->ikal", C2, w2, E2, G2)
    W2 = W2.reshape(640, 256)

    S1 = jnp.asarray(
        _np_onehot(400, 5 * 128,
                   lambda s: (s % 25) // 5 * 128 + (s // 25) * 5 + s % 5))
    N1 = jnp.asarray(_np_onehot(120, 128, lambda s: s))
    WF1 = jnp.einsum("nk,kr,nl->rl", fc1_w.astype(f), S1, N1)

    N2 = jnp.asarray(_np_onehot(84, 128, lambda s: s))
    WF2 = jnp.einsum("nk,kr,nl->rl", fc2_w.astype(f), N1, N2)
    N3 = jnp.asarray(_np_onehot(10, 128, lambda s: s))
    WF3 = jnp.einsum("nk,kr,nl->rl", fc3_w.astype(f), N2, N3)
    WF23 = jnp.concatenate([WF2, WF3], axis=0)

    pb = np.zeros((236, 2 * 1024), np.float32)
    for o in range(6):
        for g in range(8):
            pb[o, g * 128 + o * 14:g * 128 + o * 14 + 14] = 1.0
    for o in range(16):
        for a in range(2):
            pb[6 + o, 1024 + a * 128 + o * 5:1024 + a * 128 + o * 5 + 5] = 1.0
    pb[np.arange(22, 142), 1024 + 256 + np.arange(120)] = 1.0
    pb[np.arange(142, 226), 1024 + 384 + np.arange(84)] = 1.0
    pb[np.arange(226, 236), 1024 + 512 + np.arange(10)] = 1.0
    bcat = jnp.concatenate([conv1_b.astype(f), conv2_b.astype(f),
                            fc1_b.astype(f), fc2_b.astype(f),
                            fc3_b.astype(f)])
    ball = jnp.dot(bcat[None, :], jnp.asarray(pb)).reshape(2, 1024)
    return W1, W2, WF1, WF23, ball


def _net_kernel(x_ref, w1_ref, w2_ref, wf1_ref, wf23_ref, ball_ref, out_ref):
    f = _F32
    x = x_ref[...]
    w1 = w1_ref[...]
    ball = ball_ref[...]
    b1 = ball[0:1, :]

    p1_rows = []
    for c in range(7):
        y = jnp.dot(x[:, 128 * c:128 * c + 256], w1,
                    preferred_element_type=f) + b1
        m = [jnp.maximum(y[:, g * 256:g * 256 + 128],
                         y[:, g * 256 + 128:g * 256 + 256])
             for g in range(4)]
        p1_rows.append(jnp.maximum(jnp.maximum(m[0], m[1]), 0.0))
        p1_rows.append(jnp.maximum(jnp.maximum(m[2], m[3]), 0.0))
    p1 = jnp.concatenate(p1_rows, axis=1)

    w2 = w2_ref[...]
    b2 = ball[1:2, 0:256]
    cm = []
    for r in range(10):
        y2 = jnp.dot(p1[:, 128 * r:128 * r + 640], w2,
                     preferred_element_type=f) + b2
        cm.append(jnp.maximum(y2[:, :128], y2[:, 128:]))
    p2_rows = [jnp.maximum(jnp.maximum(cm[2 * q], cm[2 * q + 1]), 0.0)
               for q in range(5)]
    p2 = jnp.concatenate(p2_rows, axis=1)

    h1 = jnp.maximum(jnp.dot(p2, wf1_ref[...], preferred_element_type=f)
                     + ball[1:2, 256:384], 0.0)
    h2 = jnp.maximum(jnp.dot(h1, wf23_ref[0:128, :], preferred_element_type=f)
                     + ball[1:2, 384:512], 0.0)
    h3 = (jnp.dot(h2, wf23_ref[128:256, :], preferred_element_type=f)
          + ball[1:2, 512:640])
    out_ref[...] = h3[:, 0:10]


def _const_spec(a):
    zeros = (0,) * a.ndim
    return pl.BlockSpec(a.shape, lambda b, _z=zeros: _z)


def kernel(x, conv1_w, conv1_b, conv2_w, conv2_b, fc1_w, fc1_b,
           fc2_w, fc2_b, fc3_w, fc3_b):
    B = x.shape[0]
    tables = _build_tables(conv1_w, conv1_b, conv2_w, conv2_b,
                           fc1_w, fc1_b, fc2_w, fc2_b, fc3_w, fc3_b)
    x2 = x.astype(_F32).reshape(B, 1024)

    for bb in (1024, 256, 128, 64, 32, 16, 8, 1):
        if B % bb == 0:
            BB = bb
            break

    in_specs = [pl.BlockSpec((BB, 1024), lambda b: (b, 0))]
    in_specs += [_const_spec(a) for a in tables]

    out = pl.pallas_call(
        _net_kernel,
        out_shape=jax.ShapeDtypeStruct((B, 10), _F32),
        grid=(B // BB,),
        in_specs=in_specs,
        out_specs=pl.BlockSpec((BB, 10), lambda b: (b, 0)),
        compiler_params=pltpu.CompilerParams(
            dimension_semantics=("parallel",)),
    )(x2, *tables)
    return out
